# baseline (speedup 1.0000x reference)
import math

import jax
import jax.numpy as jnp
from jax.experimental import pallas as pl
from jax.experimental.pallas import tpu as pltpu


def _mlp_kernel(x_ref, w1_ref, b1_ref, w2_ref, b2_ref, w3_ref, b3_ref, o_ref):
    x = x_ref[...]
    h1 = jax.nn.sigmoid(
        jnp.dot(w1_ref[...], x, preferred_element_type=jnp.float32)
        + b1_ref[...])
    h2 = jax.nn.sigmoid(
        jnp.dot(w2_ref[...], h1, preferred_element_type=jnp.float32)
        + b2_ref[...])
    w3 = w3_ref[...]
    b3 = b3_ref[...]
    d = (jnp.dot(w3[1:2, :] - w3[0:1, :], h2,
                 preferred_element_type=jnp.float32)
         + (b3[1:2, :] - b3[0:1, :]))
    p1 = jax.nn.sigmoid(d)
    o_ref[...] = jnp.concatenate([1.0 - p1, p1], axis=0)


@jax.jit
def kernel(x, w1, b1, w2, b2, w3, b3):
    B, F = x.shape
    f32 = jnp.float32
    xT = x.astype(f32).T

    tb = math.gcd(B, 32768)
    grid = B // tb
    full = lambda shape: pl.BlockSpec(shape, lambda i: (0, 0))

    out = pl.pallas_call(
        _mlp_kernel,
        out_shape=jax.ShapeDtypeStruct((2, B), f32),
        grid=(grid,),
        in_specs=[
            pl.BlockSpec((F, tb), lambda i: (0, i)),
            full((12, 12)), full((12, 1)),
            full((8, 12)), full((8, 1)),
            full((2, 8)), full((2, 1)),
        ],
        out_specs=pl.BlockSpec((2, tb), lambda i: (0, i)),
        compiler_params=pltpu.CompilerParams(
            dimension_semantics=("parallel",)),
    )(xT, w1.astype(f32), b1.astype(f32), w2.astype(f32), b2.astype(f32),
      w3.astype(f32), b3.astype(f32))

    return out.T


# speedup vs baseline: 1.1230x; 1.1230x over previous
import math

import jax
import jax.numpy as jnp
from jax.experimental import pallas as pl
from jax.experimental.pallas import tpu as pltpu


def _mlp_kernel(x_ref, w1_ref, b1_ref, w2_ref, b2_ref, w3_ref, b3_ref, o_ref):
    x = x_ref[...]
    h1 = jax.nn.sigmoid(
        jnp.dot(w1_ref[...], x, preferred_element_type=jnp.float32)
        + b1_ref[...])
    h2 = jax.nn.sigmoid(
        jnp.dot(w2_ref[...], h1, preferred_element_type=jnp.float32)
        + b2_ref[...])
    w3 = w3_ref[...]
    b3 = b3_ref[...]
    d = (jnp.dot(w3[1:2, :] - w3[0:1, :], h2,
                 preferred_element_type=jnp.float32)
         + (b3[1:2, :] - b3[0:1, :]))
    p1 = jax.nn.sigmoid(d)
    o_ref[...] = jnp.concatenate([1.0 - p1, p1], axis=0)


@jax.jit
def kernel(x, w1, b1, w2, b2, w3, b3):
    B, F = x.shape
    f32 = jnp.float32
    xT = x.astype(f32).T

    tb = math.gcd(B, 65536)
    grid = B // tb
    full = lambda shape: pl.BlockSpec(shape, lambda i: (0, 0))

    out = pl.pallas_call(
        _mlp_kernel,
        out_shape=jax.ShapeDtypeStruct((2, B), f32),
        grid=(grid,),
        in_specs=[
            pl.BlockSpec((F, tb), lambda i: (0, i)),
            full((12, 12)), full((12, 1)),
            full((8, 12)), full((8, 1)),
            full((2, 8)), full((2, 1)),
        ],
        out_specs=pl.BlockSpec((2, tb), lambda i: (0, i)),
        compiler_params=pltpu.CompilerParams(
            dimension_semantics=("parallel",)),
    )(xT, w1.astype(f32), b1.astype(f32), w2.astype(f32), b2.astype(f32),
      w3.astype(f32), b3.astype(f32))

    return out.T


# speedup vs baseline: 1.1941x; 1.0634x over previous
import math

import jax
import jax.numpy as jnp
from jax.experimental import pallas as pl
from jax.experimental.pallas import tpu as pltpu

_NSTREAM = 4


def _sigmoid(z):
    return 0.5 * jnp.tanh(0.5 * z) + 0.5


def _mlp_kernel(x0_ref, x1_ref, x2_ref, x3_ref,
                w1_ref, b1_ref, w2_ref, b2_ref, w3_ref, b3_ref, o_ref):
    w1, b1 = w1_ref[...], b1_ref[...]
    w2, b2 = w2_ref[...], b2_ref[...]
    w3, b3 = w3_ref[...], b3_ref[...]
    w3d = w3[1:2, :] - w3[0:1, :]
    b3d = b3[1:2, :] - b3[0:1, :]
    tb = x0_ref.shape[1]
    for k, x_ref in enumerate((x0_ref, x1_ref, x2_ref, x3_ref)):
        x = x_ref[...]
        h1 = _sigmoid(
            jnp.dot(w1, x, preferred_element_type=jnp.float32) + b1)
        h2 = _sigmoid(
            jnp.dot(w2, h1, preferred_element_type=jnp.float32) + b2)
        d = jnp.dot(w3d, h2, preferred_element_type=jnp.float32) + b3d
        o_ref[:, k * tb:(k + 1) * tb] = _sigmoid(
            jnp.concatenate([-d, d], axis=0))


@jax.jit
def kernel(x, w1, b1, w2, b2, w3, b3):
    B, F = x.shape
    f32 = jnp.float32
    xT = x.astype(f32).T

    tb = math.gcd(B, _NSTREAM * 16384) // _NSTREAM
    grid = B // (_NSTREAM * tb)
    full = lambda shape: pl.BlockSpec(shape, lambda i: (0, 0))

    def xspec(k):
        return pl.BlockSpec((F, tb), lambda i, k=k: (0, _NSTREAM * i + k))

    out = pl.pallas_call(
        _mlp_kernel,
        out_shape=jax.ShapeDtypeStruct((2, B), f32),
        grid=(grid,),
        in_specs=[
            xspec(0), xspec(1), xspec(2), xspec(3),
            full((12, 12)), full((12, 1)),
            full((8, 12)), full((8, 1)),
            full((2, 8)), full((2, 1)),
        ],
        out_specs=pl.BlockSpec((2, _NSTREAM * tb), lambda i: (0, i)),
        compiler_params=pltpu.CompilerParams(
            dimension_semantics=("parallel",)),
    )(xT, xT, xT, xT,
      w1.astype(f32), b1.astype(f32), w2.astype(f32), b2.astype(f32),
      w3.astype(f32), b3.astype(f32))

    return out.T


# speedup vs baseline: 1.3964x; 1.1694x over previous
import math

import jax
import jax.numpy as jnp
from jax.experimental import pallas as pl
from jax.experimental.pallas import tpu as pltpu

_NSTREAM = 4


def _sigmoid(z):
    return 0.5 * jnp.tanh(0.5 * z) + 0.5


def _col(row):
    n = row.shape[1]
    eye = (jax.lax.broadcasted_iota(jnp.int32, (n, n), 0)
           == jax.lax.broadcasted_iota(jnp.int32, (n, n), 1)).astype(row.dtype)
    return jax.lax.dot_general(eye, row, (((1,), (1,)), ((), ())),
                               preferred_element_type=jnp.float32)


def _mlp_kernel(x0_ref, x1_ref, x2_ref, x3_ref,
                w1_ref, b1_ref, w2_ref, b2_ref, w3_ref, b3_ref, o_ref):
    w1, b1 = w1_ref[...], _col(b1_ref[...])
    w2, b2 = w2_ref[...], _col(b2_ref[...])
    w3, b3 = w3_ref[...], b3_ref[...]
    w3d = w3[1:2, :] - w3[0:1, :]
    b3d = b3[:, 1:2] - b3[:, 0:1]
    tb = x0_ref.shape[1]
    for k, x_ref in enumerate((x0_ref, x1_ref, x2_ref, x3_ref)):
        x = x_ref[...]
        h1 = _sigmoid(
            jnp.dot(w1, x, preferred_element_type=jnp.float32) + b1)
        h2 = _sigmoid(
            jnp.dot(w2, h1, preferred_element_type=jnp.float32) + b2)
        d = jnp.dot(w3d, h2, preferred_element_type=jnp.float32) + b3d
        o_ref[:, k * tb:(k + 1) * tb] = _sigmoid(
            jnp.concatenate([-d, d], axis=0))


@jax.jit
def kernel(x, w1, b1, w2, b2, w3, b3):
    B, F = x.shape
    f32 = jnp.float32
    xT = x.astype(f32).T

    tb = math.gcd(B, _NSTREAM * 16384) // _NSTREAM
    grid = B // (_NSTREAM * tb)
    full = lambda shape: pl.BlockSpec(shape, lambda i: (0, 0))

    def xspec(k):
        return pl.BlockSpec((F, tb), lambda i, k=k: (0, _NSTREAM * i + k))

    out = pl.pallas_call(
        _mlp_kernel,
        out_shape=jax.ShapeDtypeStruct((2, B), f32),
        grid=(grid,),
        in_specs=[
            xspec(0), xspec(1), xspec(2), xspec(3),
            full((12, 12)), full((1, 12)),
            full((8, 12)), full((1, 8)),
            full((2, 8)), full((1, 2)),
        ],
        out_specs=pl.BlockSpec((2, _NSTREAM * tb), lambda i: (0, i)),
        compiler_params=pltpu.CompilerParams(
            dimension_semantics=("parallel",)),
    )(xT, xT, xT, xT,
      w1.astype(f32), b1.astype(f32).reshape(1, 12),
      w2.astype(f32), b2.astype(f32).reshape(1, 8),
      w3.astype(f32), b3.astype(f32).reshape(1, 2))

    return out.T


# speedup vs baseline: 1.4905x; 1.0674x over previous
import math

import jax
import jax.numpy as jnp
from jax.experimental import pallas as pl
from jax.experimental.pallas import tpu as pltpu

_NSTREAM = 4


def _sigmoid(z):
    return 0.5 * jnp.tanh(0.5 * z) + 0.5


def _col(row):
    n = row.shape[1]
    eye = (jax.lax.broadcasted_iota(jnp.int32, (n, n), 0)
           == jax.lax.broadcasted_iota(jnp.int32, (n, n), 1)).astype(row.dtype)
    return jax.lax.dot_general(eye, row, (((1,), (1,)), ((), ())),
                               preferred_element_type=jnp.float32)


def _mlp_kernel(x0_ref, x1_ref, x2_ref, x3_ref,
                w1_ref, b1_ref, w2_ref, b2_ref, w3_ref, b3_ref, o_ref):
    w1, b1 = w1_ref[...], _col(b1_ref[...])
    w2, b2 = w2_ref[...], _col(b2_ref[...])
    w3, b3 = w3_ref[...], b3_ref[...]
    w3d = w3[1:2, :] - w3[0:1, :]
    b3d = b3[:, 1:2] - b3[:, 0:1]
    tb = x0_ref.shape[1]
    for k, x_ref in enumerate((x0_ref, x1_ref, x2_ref, x3_ref)):
        x = x_ref[...]
        h1 = _sigmoid(
            jnp.dot(w1, x, preferred_element_type=jnp.float32) + b1)
        h2 = _sigmoid(
            jnp.dot(w2, h1, preferred_element_type=jnp.float32) + b2)
        d = jnp.dot(w3d, h2, preferred_element_type=jnp.float32) + b3d
        o_ref[:, k * tb:(k + 1) * tb] = _sigmoid(
            jnp.concatenate([-d, d], axis=0))


@jax.jit
def kernel(x, w1, b1, w2, b2, w3, b3):
    B, F = x.shape
    f32 = jnp.float32
    xT = x.astype(f32).T

    tb = math.gcd(B, _NSTREAM * 32768) // _NSTREAM
    grid = B // (_NSTREAM * tb)
    full = lambda shape: pl.BlockSpec(shape, lambda i: (0, 0))

    def xspec(k):
        return pl.BlockSpec((F, tb), lambda i, k=k: (0, _NSTREAM * i + k))

    out = pl.pallas_call(
        _mlp_kernel,
        out_shape=jax.ShapeDtypeStruct((2, B), f32),
        grid=(grid,),
        in_specs=[
            xspec(0), xspec(1), xspec(2), xspec(3),
            full((12, 12)), full((1, 12)),
            full((8, 12)), full((1, 8)),
            full((2, 8)), full((1, 2)),
        ],
        out_specs=pl.BlockSpec((2, _NSTREAM * tb), lambda i: (0, i)),
        compiler_params=pltpu.CompilerParams(
            dimension_semantics=("parallel",)),
    )(xT, xT, xT, xT,
      w1.astype(f32), b1.astype(f32).reshape(1, 12),
      w2.astype(f32), b2.astype(f32).reshape(1, 8),
      w3.astype(f32), b3.astype(f32).reshape(1, 2))

    return out.T
